# baseline (speedup 1.0000x reference)
import functools
import jax
import jax.numpy as jnp
from jax import lax
from jax.experimental import pallas as pl
from jax.experimental.pallas import tpu as pltpu
from jax.experimental.pallas import tpu_sc as plsc

_NC, _NS = 2, 16
_NW = _NC * _NS


@functools.lru_cache(maxsize=None)
def _make_sc_kernel(B, L, D, dtype_name):
    dtype = jnp.dtype(dtype_name)
    assert B % _NW == 0
    bpw = B // _NW
    mesh = plsc.VectorSubcoreMesh(core_axis_name="c", subcore_axis_name="s")

    @functools.partial(
        pl.kernel,
        mesh=mesh,
        out_type=jax.ShapeDtypeStruct((B, L, D), dtype),
        scratch_types=[
            pltpu.VMEM_SHARED((L, D), dtype),
            pltpu.SemaphoreType.DMA,
        ],
    )
    def k(w_hbm, out_hbm, shared, sem):
        cid = lax.axis_index("c")
        sid = lax.axis_index("s")

        @pl.when(sid == 0)
        def _():
            pltpu.sync_copy(w_hbm, shared)

        plsc.subcore_barrier()
        wid = sid * _NC + cid
        base = wid * bpw
        copies = [
            pltpu.make_async_copy(shared, out_hbm.at[base + i], sem)
            for i in range(bpw)
        ]
        for c in copies:
            c.start()
        for c in copies:
            c.wait()

    return k


def kernel(x, embed_weight):
    b, l, d = x.shape
    k = _make_sc_kernel(b, l, d, str(embed_weight.dtype))
    return k(embed_weight)


# speedup vs baseline: 1.0374x; 1.0374x over previous
import functools
import jax
import jax.numpy as jnp
from jax import lax
from jax.experimental import pallas as pl
from jax.experimental.pallas import tpu as pltpu
from jax.experimental.pallas import tpu_sc as plsc

_NC, _NS = 2, 16
_NW = _NC * _NS


@functools.lru_cache(maxsize=None)
def _make_sc_kernel(B, L, D, dtype_name):
    dtype = jnp.dtype(dtype_name)
    assert B % _NW == 0
    bpw = B // _NW
    mesh = plsc.VectorSubcoreMesh(core_axis_name="c", subcore_axis_name="s")

    @functools.partial(
        pl.kernel,
        mesh=mesh,
        out_type=jax.ShapeDtypeStruct((B, L, D), dtype),
        scratch_types=[
            pltpu.VMEM((L, D), dtype),
            pltpu.SemaphoreType.DMA,
        ],
    )
    def k(w_hbm, out_hbm, local, sem):
        cid = lax.axis_index("c")
        sid = lax.axis_index("s")
        pltpu.sync_copy(w_hbm, local)
        wid = sid * _NC + cid
        base = wid * bpw
        copies = [
            pltpu.make_async_copy(local, out_hbm.at[base + i], sem)
            for i in range(bpw)
        ]
        for c in copies:
            c.start()
        for c in copies:
            c.wait()

    return k


def kernel(x, embed_weight):
    b, l, d = x.shape
    k = _make_sc_kernel(b, l, d, str(embed_weight.dtype))
    return k(embed_weight)
